# baseline (device time: 27389 ns/iter reference)
import jax
import jax.numpy as jnp
from jax import lax
from jax.experimental import pallas as pl
from jax.experimental.pallas import tpu as pltpu


def kernel(x, router, W1, W2):
    t_loc, d = x.shape
    e_loc, _, f = W1.shape

    def body(x_ref, r_ref, w1_ref, w2_ref, out_ref,
             xsend, xrec, rrec, wsend, wrec, ysend, yrec,
             send_sems, recv_sems):
        my_x = lax.axis_index("x")
        my_y = lax.axis_index("y")
        my_z = lax.axis_index("z")
        partner = (my_x, my_y, 1 - my_z)
        z0 = my_z == 0

        def rdma(src, dst, i):
            return pltpu.make_async_remote_copy(
                src_ref=src, dst_ref=dst,
                send_sem=send_sems.at[i], recv_sem=recv_sems.at[i],
                device_id=partner, device_id_type=pl.DeviceIdType.MESH,
            )

        barrier = pltpu.get_barrier_semaphore()
        pl.semaphore_signal(barrier, inc=1, device_id=partner,
                            device_id_type=pl.DeviceIdType.MESH)
        pl.semaphore_wait(barrier, 1)

        xsend[...] = x_ref[...].astype(jnp.bfloat16)
        rdma_x = rdma(xsend, xrec, 1)
        rdma_x.start()
        rdma_r = rdma(r_ref, rrec, 0)
        rdma_r.start()

        w1b = [w1_ref[j].astype(jnp.bfloat16) for j in range(e_loc)]
        w2b = [w2_ref[j].astype(jnp.bfloat16) for j in range(e_loc)]

        rdma_r.wait()
        g = jnp.dot(x_ref[...],
                    jnp.concatenate([r_ref[...], rrec[...]], axis=1),
                    preferred_element_type=jnp.float32)
        cols = [
            jnp.where(z0, g[:, 0:1], g[:, 2:3]),
            jnp.where(z0, g[:, 1:2], g[:, 3:4]),
            jnp.where(z0, g[:, 2:3], g[:, 0:1]),
            jnp.where(z0, g[:, 3:4], g[:, 1:2]),
        ]
        m = jnp.maximum(jnp.maximum(cols[0], cols[1]),
                        jnp.maximum(cols[2], cols[3]))
        w = []
        for e in range(4):
            rank = sum(
                jnp.where(cols[o] >= cols[e] if o < e else cols[o] > cols[e],
                          1, 0)
                for o in range(4) if o != e)
            w.append(jnp.where(rank < 2, jnp.exp(cols[e] - m), 0.0))
        denom = w[0] + w[1] + w[2] + w[3]
        w = [wi / denom for wi in w]
        wmine = [jnp.where(z0, w[j], w[2 + j]) for j in range(e_loc)]
        wsend[...] = jnp.concatenate(
            [jnp.where(z0, w[2 + j], w[j]) for j in range(e_loc)], axis=1)
        rdma_w = rdma(wsend, wrec, 2)
        rdma_w.start()

        rdma_x.wait()
        rdma_w.wait()
        xr = xrec[...]
        acc_rec = jnp.zeros((t_loc, d), jnp.float32)
        for j in range(e_loc):
            h = jnp.dot(xr, w1b[j], preferred_element_type=jnp.float32)
            h = jnp.maximum(h, 0.0).astype(jnp.bfloat16)
            acc_rec += jnp.dot(h, w2b[j], preferred_element_type=jnp.float32) \
                * wrec[:, j:j + 1]
        ysend[...] = acc_rec.astype(jnp.bfloat16)
        rdma_y = rdma(ysend, yrec, 3)
        rdma_y.start()

        xl = xsend[...]
        acc_loc = jnp.zeros((t_loc, d), jnp.float32)
        for j in range(e_loc):
            h = jnp.dot(xl, w1b[j], preferred_element_type=jnp.float32)
            h = jnp.maximum(h, 0.0).astype(jnp.bfloat16)
            acc_loc += jnp.dot(h, w2b[j], preferred_element_type=jnp.float32) \
                * wmine[j]

        rdma_y.wait()
        out_ref[...] = acc_loc + yrec[...].astype(jnp.float32)

    return pl.pallas_call(
        body,
        out_shape=jax.ShapeDtypeStruct((t_loc, d), jnp.float32),
        in_specs=[pl.BlockSpec(memory_space=pltpu.VMEM)] * 4,
        out_specs=pl.BlockSpec(memory_space=pltpu.VMEM),
        scratch_shapes=[
            pltpu.VMEM((t_loc, d), jnp.bfloat16),
            pltpu.VMEM((t_loc, d), jnp.bfloat16),
            pltpu.VMEM((d, e_loc), jnp.float32),
            pltpu.VMEM((t_loc, e_loc), jnp.float32),
            pltpu.VMEM((t_loc, e_loc), jnp.float32),
            pltpu.VMEM((t_loc, d), jnp.bfloat16),
            pltpu.VMEM((t_loc, d), jnp.bfloat16),
            pltpu.SemaphoreType.DMA((4,)),
            pltpu.SemaphoreType.DMA((4,)),
        ],
        compiler_params=pltpu.CompilerParams(collective_id=0),
    )(x, router, W1, W2)


# device time: 26847 ns/iter; 1.0202x vs baseline; 1.0202x over previous
import jax
import jax.numpy as jnp
from jax import lax
from jax.experimental import pallas as pl
from jax.experimental.pallas import tpu as pltpu


def kernel(x, router, W1, W2):
    t_loc, d = x.shape
    e_loc, _, f = W1.shape
    n_ch = 2
    h_ch = t_loc // n_ch

    def body(x_ref, r_ref, w1_ref, w2_ref, out_ref,
             xv, w1v, w2v, outv,
             xsend, xrec, rrec, wsend, wrec, ysend, yrec,
             dma_sems, send_sems, recv_sems):
        my_x = lax.axis_index("x")
        my_y = lax.axis_index("y")
        my_z = lax.axis_index("z")
        partner = (my_x, my_y, 1 - my_z)
        z0 = my_z == 0

        def rdma(src, dst, i):
            return pltpu.make_async_remote_copy(
                src_ref=src, dst_ref=dst,
                send_sem=send_sems.at[i], recv_sem=recv_sems.at[i],
                device_id=partner, device_id_type=pl.DeviceIdType.MESH,
            )

        cp_x = pltpu.make_async_copy(x_ref, xv, dma_sems.at[0])
        cp_x.start()
        cp_w1 = pltpu.make_async_copy(w1_ref, w1v, dma_sems.at[1])
        cp_w1.start()
        cp_w2 = pltpu.make_async_copy(w2_ref, w2v, dma_sems.at[2])
        cp_w2.start()

        barrier = pltpu.get_barrier_semaphore()
        pl.semaphore_signal(barrier, inc=1, device_id=partner,
                            device_id_type=pl.DeviceIdType.MESH)
        pl.semaphore_wait(barrier, 1)

        cp_x.wait()
        xf = xv[...]
        xsend[...] = xf.reshape(n_ch, h_ch, d).astype(jnp.bfloat16)
        rx = []
        for c in range(n_ch):
            r = rdma(xsend.at[c], xrec.at[c], 2 + c)
            r.start()
            rx.append(r)
        rr = rdma(r_ref, rrec, 0)
        rr.start()

        cp_w1.wait()
        cp_w2.wait()
        w1b = [w1v[j].astype(jnp.bfloat16) for j in range(e_loc)]
        w2b = [w2v[j].astype(jnp.bfloat16) for j in range(e_loc)]

        rr.wait()
        g = jnp.dot(xf, jnp.concatenate([r_ref[...], rrec[...]], axis=1),
                    preferred_element_type=jnp.float32)
        cols = [
            jnp.where(z0, g[:, 0:1], g[:, 2:3]),
            jnp.where(z0, g[:, 1:2], g[:, 3:4]),
            jnp.where(z0, g[:, 2:3], g[:, 0:1]),
            jnp.where(z0, g[:, 3:4], g[:, 1:2]),
        ]
        m = jnp.maximum(jnp.maximum(cols[0], cols[1]),
                        jnp.maximum(cols[2], cols[3]))
        w = []
        for e in range(4):
            rank = sum(
                jnp.where(cols[o] >= cols[e] if o < e else cols[o] > cols[e],
                          1, 0)
                for o in range(4) if o != e)
            w.append(jnp.where(rank < 2, jnp.exp(cols[e] - m), 0.0))
        denom = w[0] + w[1] + w[2] + w[3]
        w = [wi / denom for wi in w]
        wmine = [jnp.where(z0, w[j], w[2 + j]) for j in range(e_loc)]
        wsend[...] = jnp.concatenate(
            [jnp.where(z0, w[2 + j], w[j]) for j in range(e_loc)], axis=1)
        rw = rdma(wsend, wrec, 1)
        rw.start()

        ry = []
        for c in range(n_ch):
            rx[c].wait()
            xr = xrec[c]
            ys = []
            for j in range(e_loc):
                hcj = jnp.dot(xr, w1b[j], preferred_element_type=jnp.float32)
                hcj = jnp.maximum(hcj, 0.0).astype(jnp.bfloat16)
                ys.append(jnp.dot(hcj, w2b[j],
                                  preferred_element_type=jnp.float32))
            if c == 0:
                rw.wait()
            wr = wrec[c * h_ch:(c + 1) * h_ch, :]
            ysend[c, :, :] = (ys[0] * wr[:, 0:1]
                              + ys[1] * wr[:, 1:2]).astype(jnp.bfloat16)
            r = rdma(ysend.at[c], yrec.at[c], 4 + c)
            r.start()
            ry.append(r)

        xl = xsend[...].reshape(t_loc, d)
        acc_loc = jnp.zeros((t_loc, d), jnp.float32)
        for j in range(e_loc):
            hl = jnp.dot(xl, w1b[j], preferred_element_type=jnp.float32)
            hl = jnp.maximum(hl, 0.0).astype(jnp.bfloat16)
            acc_loc += jnp.dot(hl, w2b[j], preferred_element_type=jnp.float32) \
                * wmine[j]

        for c in range(n_ch):
            ry[c].wait()
        outv[...] = acc_loc + yrec[...].reshape(t_loc, d).astype(jnp.float32)
        cp_out = pltpu.make_async_copy(outv, out_ref, dma_sems.at[3])
        cp_out.start()
        cp_out.wait()

    return pl.pallas_call(
        body,
        out_shape=jax.ShapeDtypeStruct((t_loc, d), jnp.float32),
        in_specs=[
            pl.BlockSpec(memory_space=pl.ANY),
            pl.BlockSpec(memory_space=pltpu.VMEM),
            pl.BlockSpec(memory_space=pl.ANY),
            pl.BlockSpec(memory_space=pl.ANY),
        ],
        out_specs=pl.BlockSpec(memory_space=pl.ANY),
        scratch_shapes=[
            pltpu.VMEM((t_loc, d), jnp.float32),
            pltpu.VMEM((e_loc, d, f), jnp.float32),
            pltpu.VMEM((e_loc, f, d), jnp.float32),
            pltpu.VMEM((t_loc, d), jnp.float32),
            pltpu.VMEM((n_ch, h_ch, d), jnp.bfloat16),
            pltpu.VMEM((n_ch, h_ch, d), jnp.bfloat16),
            pltpu.VMEM((d, e_loc), jnp.float32),
            pltpu.VMEM((t_loc, e_loc), jnp.float32),
            pltpu.VMEM((t_loc, e_loc), jnp.float32),
            pltpu.VMEM((n_ch, h_ch, d), jnp.bfloat16),
            pltpu.VMEM((n_ch, h_ch, d), jnp.bfloat16),
            pltpu.SemaphoreType.DMA((4,)),
            pltpu.SemaphoreType.DMA((6,)),
            pltpu.SemaphoreType.DMA((6,)),
        ],
        compiler_params=pltpu.CompilerParams(collective_id=0),
    )(x, router, W1, W2)


# device time: 18776 ns/iter; 1.4587x vs baseline; 1.4299x over previous
import jax
import jax.numpy as jnp
from jax import lax
from jax.experimental import pallas as pl
from jax.experimental.pallas import tpu as pltpu


def kernel(x, router, W1, W2):
    t_loc, d = x.shape
    e_loc, _, f = W1.shape
    n_ch = 2
    h_ch = t_loc // n_ch

    def body(x_ref, r_ref, w1_ref, w2_ref, out_ref,
             xsend, xrec, rsend, rrec, ysend, yrec,
             send_sems, recv_sems):
        my_x = lax.axis_index("x")
        my_y = lax.axis_index("y")
        my_z = lax.axis_index("z")
        partner = (my_x, my_y, 1 - my_z)
        z0 = my_z == 0

        def rdma(src, dst, i):
            return pltpu.make_async_remote_copy(
                src_ref=src, dst_ref=dst,
                send_sem=send_sems.at[i], recv_sem=recv_sems.at[i],
                device_id=partner, device_id_type=pl.DeviceIdType.MESH,
            )

        barrier = pltpu.get_barrier_semaphore()
        pl.semaphore_signal(barrier, inc=1, device_id=partner,
                            device_id_type=pl.DeviceIdType.MESH)
        rsend[...] = r_ref[...].T.astype(jnp.bfloat16)
        xsend[...] = x_ref[...].reshape(n_ch, h_ch, d).astype(jnp.bfloat16)
        pl.semaphore_wait(barrier, 1)

        rr = rdma(rsend, rrec, 0)
        rr.start()
        rx = []
        for c in range(n_ch):
            r = rdma(xsend.at[c], xrec.at[c], 1 + c)
            r.start()
            rx.append(r)

        w1b = [w1_ref[j].astype(jnp.bfloat16) for j in range(e_loc)]
        w2b = [w2_ref[j].astype(jnp.bfloat16) for j in range(e_loc)]

        def gates(xb):
            gm = lax.dot_general(xb, rsend[...],
                                 dimension_numbers=(((1,), (1,)), ((), ())),
                                 preferred_element_type=jnp.float32)
            go = lax.dot_general(xb, rrec[...],
                                 dimension_numbers=(((1,), (1,)), ((), ())),
                                 preferred_element_type=jnp.float32)
            cols = [
                jnp.where(z0, gm[:, 0:1], go[:, 0:1]),
                jnp.where(z0, gm[:, 1:2], go[:, 1:2]),
                jnp.where(z0, go[:, 0:1], gm[:, 0:1]),
                jnp.where(z0, go[:, 1:2], gm[:, 1:2]),
            ]
            m = jnp.maximum(jnp.maximum(cols[0], cols[1]),
                            jnp.maximum(cols[2], cols[3]))
            w = []
            for e in range(4):
                rank = sum(
                    jnp.where(
                        cols[o] >= cols[e] if o < e else cols[o] > cols[e],
                        1, 0)
                    for o in range(4) if o != e)
                w.append(jnp.where(rank < 2, jnp.exp(cols[e] - m), 0.0))
            denom = w[0] + w[1] + w[2] + w[3]
            return [jnp.where(z0, w[j], w[2 + j]) / denom
                    for j in range(e_loc)]

        rr.wait()

        ry = []
        for c in range(n_ch):
            rx[c].wait()
            xr = xrec[c]
            wrs = gates(xr)
            acc = jnp.zeros((h_ch, d), jnp.float32)
            for j in range(e_loc):
                hcj = jnp.dot(xr, w1b[j], preferred_element_type=jnp.float32)
                hcj = jnp.maximum(hcj, 0.0).astype(jnp.bfloat16)
                acc += jnp.dot(hcj, w2b[j],
                               preferred_element_type=jnp.float32) * wrs[j]
            ysend[c, :, :] = acc.astype(jnp.bfloat16)
            r = rdma(ysend.at[c], yrec.at[c], 1 + n_ch + c)
            r.start()
            ry.append(r)

        xl = xsend[...].reshape(t_loc, d)
        wls = gates(xl)
        acc_loc = jnp.zeros((t_loc, d), jnp.float32)
        for j in range(e_loc):
            hl = jnp.dot(xl, w1b[j], preferred_element_type=jnp.float32)
            hl = jnp.maximum(hl, 0.0).astype(jnp.bfloat16)
            acc_loc += jnp.dot(hl, w2b[j], preferred_element_type=jnp.float32) \
                * wls[j]

        for c in range(n_ch):
            ry[c].wait()
        out_ref[...] = acc_loc + yrec[...].reshape(t_loc, d).astype(jnp.float32)

    return pl.pallas_call(
        body,
        out_shape=jax.ShapeDtypeStruct((t_loc, d), jnp.float32),
        in_specs=[pl.BlockSpec(memory_space=pltpu.VMEM)] * 4,
        out_specs=pl.BlockSpec(memory_space=pltpu.VMEM),
        scratch_shapes=[
            pltpu.VMEM((n_ch, h_ch, d), jnp.bfloat16),
            pltpu.VMEM((n_ch, h_ch, d), jnp.bfloat16),
            pltpu.VMEM((e_loc, d), jnp.bfloat16),
            pltpu.VMEM((e_loc, d), jnp.bfloat16),
            pltpu.VMEM((n_ch, h_ch, d), jnp.bfloat16),
            pltpu.VMEM((n_ch, h_ch, d), jnp.bfloat16),
            pltpu.SemaphoreType.DMA((1 + 2 * n_ch,)),
            pltpu.SemaphoreType.DMA((1 + 2 * n_ch,)),
        ],
        compiler_params=pltpu.CompilerParams(collective_id=0),
    )(x, router, W1, W2)


# device time: 18764 ns/iter; 1.4597x vs baseline; 1.0006x over previous
import jax
import jax.numpy as jnp
from jax import lax
from jax.experimental import pallas as pl
from jax.experimental.pallas import tpu as pltpu


def kernel(x, router, W1, W2):
    t_loc, d = x.shape
    e_loc, _, f = W1.shape
    n_ch = 2
    h_ch = t_loc // n_ch

    def body(x_ref, r_ref, w1_ref, w2_ref, out_ref,
             xsend, xrec, rsend, rrec, ysend, yrec,
             send_sems, recv_sems):
        my_x = lax.axis_index("x")
        my_y = lax.axis_index("y")
        my_z = lax.axis_index("z")
        partner = (my_x, my_y, 1 - my_z)
        z0 = my_z == 0

        def rdma(src, dst, i):
            return pltpu.make_async_remote_copy(
                src_ref=src, dst_ref=dst,
                send_sem=send_sems.at[i], recv_sem=recv_sems.at[i],
                device_id=partner, device_id_type=pl.DeviceIdType.MESH,
            )

        barrier = pltpu.get_barrier_semaphore()
        pl.semaphore_signal(barrier, inc=1, device_id=partner,
                            device_id_type=pl.DeviceIdType.MESH)
        rsend[...] = r_ref[...].T.astype(jnp.bfloat16)
        xsend[...] = x_ref[...].reshape(n_ch, h_ch, d).astype(jnp.bfloat16)
        pl.semaphore_wait(barrier, 1)

        rr = rdma(rsend, rrec, 0)
        rr.start()
        rx = []
        for c in range(n_ch):
            r = rdma(xsend.at[c], xrec.at[c], 1 + c)
            r.start()
            rx.append(r)

        w1b = [w1_ref[j].astype(jnp.bfloat16) for j in range(e_loc)]
        w2b = [w2_ref[j].astype(jnp.bfloat16) for j in range(e_loc)]

        def gates(xb):
            gm = lax.dot_general(xb, rsend[...],
                                 dimension_numbers=(((1,), (1,)), ((), ())),
                                 preferred_element_type=jnp.float32)
            go = lax.dot_general(xb, rrec[...],
                                 dimension_numbers=(((1,), (1,)), ((), ())),
                                 preferred_element_type=jnp.float32)
            cols = [
                jnp.where(z0, gm[:, 0:1], go[:, 0:1]),
                jnp.where(z0, gm[:, 1:2], go[:, 1:2]),
                jnp.where(z0, go[:, 0:1], gm[:, 0:1]),
                jnp.where(z0, go[:, 1:2], gm[:, 1:2]),
            ]
            m = jnp.maximum(jnp.maximum(cols[0], cols[1]),
                            jnp.maximum(cols[2], cols[3]))
            w = []
            for e in range(4):
                rank = sum(
                    jnp.where(
                        cols[o] >= cols[e] if o < e else cols[o] > cols[e],
                        1, 0)
                    for o in range(4) if o != e)
                w.append(jnp.where(rank < 2, jnp.exp(cols[e] - m), 0.0))
            denom = w[0] + w[1] + w[2] + w[3]
            return [jnp.where(z0, w[j], w[2 + j]) / denom
                    for j in range(e_loc)]

        rr.wait()

        ry = []
        for c in range(n_ch):
            rx[c].wait()
            xr = xrec[c]
            wrs = gates(xr)
            acc = jnp.zeros((h_ch, d), jnp.float32)
            for j in range(e_loc):
                hcj = jnp.dot(xr, w1b[j], preferred_element_type=jnp.float32)
                hcj = jnp.maximum(hcj, 0.0).astype(jnp.bfloat16)
                acc += jnp.dot(hcj, w2b[j],
                               preferred_element_type=jnp.float32) * wrs[j]
            ysend[c, :, :] = acc.astype(jnp.bfloat16)
            r = rdma(ysend.at[c], yrec.at[c], 1 + n_ch + c)
            r.start()
            ry.append(r)

        xl = xsend[...].reshape(t_loc, d)
        wls = gates(xl)
        acc_loc = jnp.zeros((t_loc, d), jnp.float32)
        for j in range(e_loc):
            hl = jnp.dot(xl, w1b[j], preferred_element_type=jnp.float32)
            hl = jnp.maximum(hl, 0.0).astype(jnp.bfloat16)
            acc_loc += jnp.dot(hl, w2b[j], preferred_element_type=jnp.float32) \
                * wls[j]

        for c in range(n_ch):
            ry[c].wait()
            rows = slice(c * h_ch, (c + 1) * h_ch)
            out_ref[rows, :] = acc_loc[rows, :] \
                + yrec[c].astype(jnp.float32)

    return pl.pallas_call(
        body,
        out_shape=jax.ShapeDtypeStruct((t_loc, d), jnp.float32),
        in_specs=[pl.BlockSpec(memory_space=pltpu.VMEM)] * 4,
        out_specs=pl.BlockSpec(memory_space=pltpu.VMEM),
        scratch_shapes=[
            pltpu.VMEM((n_ch, h_ch, d), jnp.bfloat16),
            pltpu.VMEM((n_ch, h_ch, d), jnp.bfloat16),
            pltpu.VMEM((e_loc, d), jnp.bfloat16),
            pltpu.VMEM((e_loc, d), jnp.bfloat16),
            pltpu.VMEM((n_ch, h_ch, d), jnp.bfloat16),
            pltpu.VMEM((n_ch, h_ch, d), jnp.bfloat16),
            pltpu.SemaphoreType.DMA((1 + 2 * n_ch,)),
            pltpu.SemaphoreType.DMA((1 + 2 * n_ch,)),
        ],
        compiler_params=pltpu.CompilerParams(collective_id=0),
    )(x, router, W1, W2)
